# baseline (speedup 1.0000x reference)
import functools

import jax
import jax.numpy as jnp
from jax import lax
from jax.experimental import pallas as pl
from jax.experimental.pallas import tpu as pltpu
from jax.experimental.pallas import tpu_sc as plsc

N = 10000
NP = 10240
H = 128
E = 320000
LANE = 128
ROWS = 2560
EP = ROWS * LANE
NC, NS = 2, 16
NW = NC * NS
RPW = ROWS // NW
IDXB = 8
CNTW = 128
ROWS_PER_SUB = NP // NS


def _sc_agg_body(ea, eb, z_hbm, zeros_hbm, agg_hbm,
                 agg_sh, idx_g, idx_s, rows_v, sem0, sem1):
    sems = (sem0, sem1)
    c = lax.axis_index("c")
    s = lax.axis_index("s")
    stripe = pl.ds(s * ROWS_PER_SUB, ROWS_PER_SUB)

    pltpu.sync_copy(zeros_hbm, agg_sh.at[stripe])
    plsc.subcore_barrier()

    base_row = c * (ROWS // NC) + s * RPW

    def _scatter(j):
        pltpu.sync_copy(rows_v.at[pl.ds((j % 2) * LANE, LANE)],
                        agg_sh.at[idx_s.at[j]], add=True)

    def _run_pass(gather_e, scatter_e):
        def _block(b, _):
            r0 = base_row + b * IDXB
            pltpu.sync_copy(gather_e.at[pl.ds(r0, IDXB)], idx_g)
            pltpu.sync_copy(scatter_e.at[pl.ds(r0, IDXB)], idx_s)
            cps = [None, None]
            for j in range(IDXB):
                cp = pltpu.make_async_copy(
                    z_hbm.at[idx_g.at[j]],
                    rows_v.at[pl.ds((j % 2) * LANE, LANE)],
                    sems[j % 2],
                )
                cp.start()
                cps[j % 2] = cp
                if j > 0:
                    cps[(j - 1) % 2].wait()
                    _scatter(j - 1)
            cps[(IDXB - 1) % 2].wait()
            _scatter(IDXB - 1)
            return 0

        lax.fori_loop(0, RPW // IDXB, _block, 0)

    _run_pass(ea, eb)
    _run_pass(eb, ea)

    plsc.subcore_barrier()

    pltpu.sync_copy(agg_sh.at[stripe], agg_hbm.at[c, stripe])


def _make_sc_agg():
    mesh = plsc.VectorSubcoreMesh(core_axis_name="c", subcore_axis_name="s")
    return pl.kernel(
        _sc_agg_body,
        out_type=jax.ShapeDtypeStruct((NC, NP, H), jnp.float32),
        mesh=mesh,
        scratch_types=[
            pltpu.VMEM_SHARED((NP, H), jnp.float32),
            pltpu.VMEM((IDXB, LANE), jnp.int32),
            pltpu.VMEM((IDXB, LANE), jnp.int32),
            pltpu.VMEM((2 * LANE, H), jnp.float32),
            pltpu.SemaphoreType.DMA,
            pltpu.SemaphoreType.DMA,
        ],
        name="sc_edge_agg",
    )


CHC = 8


def _sc_cnt_body(ea, eb, zeros_cnt_hbm, ones_hbm, cnt_hbm,
                 cnt_sh, idx_s, ones_v):
    c = lax.axis_index("c")
    s = lax.axis_index("s")
    stripe = pl.ds(s * ROWS_PER_SUB, ROWS_PER_SUB)

    pltpu.sync_copy(zeros_cnt_hbm, cnt_sh.at[stripe])
    pltpu.sync_copy(ones_hbm, ones_v)
    plsc.subcore_barrier()

    base_row = c * (ROWS // NC) + s * RPW

    def _run_pass(scatter_e):
        def _chunk(i, _):
            r0 = base_row + i * CHC
            pltpu.sync_copy(scatter_e.at[pl.ds(r0, CHC)], idx_s)
            for j in range(CHC):
                pltpu.sync_copy(ones_v, cnt_sh.at[idx_s.at[j]], add=True)
            return 0

        lax.fori_loop(0, RPW // CHC, _chunk, 0)

    _run_pass(eb)
    _run_pass(ea)

    plsc.subcore_barrier()
    pltpu.sync_copy(cnt_sh.at[stripe], cnt_hbm.at[c, stripe])


def _make_sc_cnt():
    mesh = plsc.VectorSubcoreMesh(core_axis_name="c", subcore_axis_name="s")
    return pl.kernel(
        _sc_cnt_body,
        out_type=jax.ShapeDtypeStruct((NC, NP, CNTW), jnp.float32),
        mesh=mesh,
        scratch_types=[
            pltpu.VMEM_SHARED((NP, CNTW), jnp.float32),
            pltpu.VMEM((CHC, LANE), jnp.int32),
            pltpu.VMEM((LANE, CNTW), jnp.float32),
        ],
        name="sc_edge_cnt",
    )


BLK = 1024
GRID = NP // BLK


def _tc_pre_body(x_ref, wr_ref, b_ref, r_ref):
    r_ref[...] = (jnp.dot(x_ref[...], wr_ref[...],
                          preferred_element_type=jnp.float32,
                          precision=lax.Precision.HIGHEST)
                  + b_ref[...])


def _tc_mid_body(aggp_ref, inv_ref, r0_ref, wl_ref, wr_ref, b_ref,
                 x1_ref, r_ref):
    mean = (aggp_ref[0] + aggp_ref[1]) * inv_ref[...]
    x1 = jnp.maximum(
        jnp.dot(mean, wl_ref[...], preferred_element_type=jnp.float32,
                precision=lax.Precision.HIGHEST) + r0_ref[...], 0.0)
    x1_ref[...] = x1
    r_ref[...] = (jnp.dot(x1, wr_ref[...], preferred_element_type=jnp.float32,
                          precision=lax.Precision.HIGHEST)
                  + b_ref[...])


def _tc_out_body(aggp_ref, inv_ref, r_ref, wl_ref, o_ref):
    mean = (aggp_ref[0] + aggp_ref[1]) * inv_ref[...]
    o_ref[...] = jnp.maximum(
        jnp.dot(mean, wl_ref[...], preferred_element_type=jnp.float32,
                precision=lax.Precision.HIGHEST) + r_ref[...], 0.0)


_row_spec = pl.BlockSpec((BLK, H), lambda i: (i, 0))
_mat_spec = pl.BlockSpec((H, H), lambda i: (0, 0))
_bias_spec = pl.BlockSpec((1, H), lambda i: (0, 0))
_pair_spec = pl.BlockSpec((NC, BLK, H), lambda i: (0, i, 0))

_tc_pre = pl.pallas_call(
    _tc_pre_body,
    grid=(GRID,),
    in_specs=[_row_spec, _mat_spec, _bias_spec],
    out_specs=_row_spec,
    out_shape=jax.ShapeDtypeStruct((NP, H), jnp.float32),
)

_tc_mid = pl.pallas_call(
    _tc_mid_body,
    grid=(GRID,),
    in_specs=[_pair_spec, _row_spec, _row_spec, _mat_spec, _mat_spec, _bias_spec],
    out_specs=[_row_spec, _row_spec],
    out_shape=[jax.ShapeDtypeStruct((NP, H), jnp.float32)] * 2,
)

_tc_out = pl.pallas_call(
    _tc_out_body,
    grid=(GRID,),
    in_specs=[_pair_spec, _row_spec, _row_spec, _mat_spec],
    out_specs=_row_spec,
    out_shape=jax.ShapeDtypeStruct((NP, H), jnp.float32),
)


def kernel(edge_index, user_table, item_table, Wl0, bl0, Wr0, Wl1, bl1, Wr1):
    x = jnp.concatenate(
        [user_table, item_table,
         jnp.zeros((NP - N, H), jnp.float32)], axis=0)

    pad_idx = (jnp.arange(EP - E, dtype=jnp.int32) % (NP - N)) + N
    pad = jnp.broadcast_to(pad_idx, (2, EP - E))
    ep = jnp.concatenate([edge_index, pad], axis=1).reshape(2, ROWS, LANE)
    ea, eb = ep[0], ep[1]

    bl0_2 = bl0.reshape(1, H)
    bl1_2 = bl1.reshape(1, H)

    sc_agg = _make_sc_agg()
    sc_cnt = _make_sc_cnt()
    zeros_c = jnp.zeros((ROWS_PER_SUB, H), jnp.float32)
    zeros_cnt = jnp.zeros((ROWS_PER_SUB, CNTW), jnp.float32)
    ones_c = jnp.ones((LANE, CNTW), jnp.float32)

    cntp = sc_cnt(ea, eb, zeros_cnt, ones_c)
    cnt = cntp[0, :, 0] + cntp[1, :, 0]
    inv2d = jnp.broadcast_to((1.0 / jnp.maximum(cnt, 1.0))[:, None], (NP, H))

    aggp0 = sc_agg(ea, eb, x, zeros_c)
    r0 = _tc_pre(x, Wr0, bl0_2)

    x1, r1 = _tc_mid(aggp0, inv2d, r0, Wl0, Wr1, bl1_2)
    aggp1 = sc_agg(ea, eb, x1, zeros_c)
    out = _tc_out(aggp1, inv2d, r1, Wl1)

    return out[:N]


# speedup vs baseline: 1.0052x; 1.0052x over previous
import functools

import jax
import jax.numpy as jnp
from jax import lax
from jax.experimental import pallas as pl
from jax.experimental.pallas import tpu as pltpu
from jax.experimental.pallas import tpu_sc as plsc

N = 10000
NP = 10240
H = 128
E = 320000
LANE = 128
ROWS = 2560
EP = ROWS * LANE
NC, NS = 2, 16
NW = NC * NS
RPW = ROWS // NW
IDXB = 8
CNTW = 128
ROWS_PER_SUB = NP // NS


def _sc_agg_body(ea, eb, z_hbm, zeros_hbm, agg_hbm,
                 agg_sh, idx_g, idx_s, rows_v, sem0, sem1):
    sems = (sem0, sem1)
    c = lax.axis_index("c")
    s = lax.axis_index("s")
    stripe = pl.ds(s * ROWS_PER_SUB, ROWS_PER_SUB)

    pltpu.sync_copy(zeros_hbm, agg_sh.at[stripe])
    plsc.subcore_barrier()

    base_row = c * (ROWS // NC) + s * RPW

    def _scatter(j):
        pltpu.sync_copy(rows_v.at[pl.ds((j % 2) * LANE, LANE)],
                        agg_sh.at[idx_s.at[j]], add=True)

    def _run_pass(gather_e, scatter_e):
        def _block(b, _):
            r0 = base_row + b * IDXB
            pltpu.sync_copy(gather_e.at[pl.ds(r0, IDXB)], idx_g)
            pltpu.sync_copy(scatter_e.at[pl.ds(r0, IDXB)], idx_s)
            cps = [None, None]
            for j in range(IDXB):
                cp = pltpu.make_async_copy(
                    z_hbm.at[idx_g.at[j]],
                    rows_v.at[pl.ds((j % 2) * LANE, LANE)],
                    sems[j % 2],
                )
                cp.start()
                cps[j % 2] = cp
                if j > 0:
                    cps[(j - 1) % 2].wait()
                    _scatter(j - 1)
            cps[(IDXB - 1) % 2].wait()
            _scatter(IDXB - 1)
            return 0

        lax.fori_loop(0, RPW // IDXB, _block, 0)

    _run_pass(ea, eb)
    _run_pass(eb, ea)

    plsc.subcore_barrier()

    pltpu.sync_copy(agg_sh.at[stripe], agg_hbm.at[c, stripe])


def _make_sc_agg():
    mesh = plsc.VectorSubcoreMesh(core_axis_name="c", subcore_axis_name="s")
    return pl.kernel(
        _sc_agg_body,
        out_type=jax.ShapeDtypeStruct((NC, NP, H), jnp.float32),
        mesh=mesh,
        scratch_types=[
            pltpu.VMEM_SHARED((NP, H), jnp.float32),
            pltpu.VMEM((IDXB, LANE), jnp.int32),
            pltpu.VMEM((IDXB, LANE), jnp.int32),
            pltpu.VMEM((2 * LANE, H), jnp.float32),
            pltpu.SemaphoreType.DMA,
            pltpu.SemaphoreType.DMA,
        ],
        name="sc_edge_agg",
    )


CHC = 8


def _sc_cnt_body(ea, eb, zeros_cnt_hbm, ones_hbm, cnt_hbm,
                 cnt_sh, idx_s, ones_v):
    c = lax.axis_index("c")
    s = lax.axis_index("s")
    stripe = pl.ds(s * ROWS_PER_SUB, ROWS_PER_SUB)

    pltpu.sync_copy(zeros_cnt_hbm, cnt_sh.at[stripe])
    pltpu.sync_copy(ones_hbm, ones_v)
    plsc.subcore_barrier()

    base_row = c * (ROWS // NC) + s * RPW

    def _run_pass(scatter_e):
        def _chunk(i, _):
            r0 = base_row + i * CHC
            pltpu.sync_copy(scatter_e.at[pl.ds(r0, CHC)], idx_s)
            for j in range(CHC):
                pltpu.sync_copy(ones_v, cnt_sh.at[idx_s.at[j]], add=True)
            return 0

        lax.fori_loop(0, RPW // CHC, _chunk, 0)

    _run_pass(eb)
    _run_pass(ea)

    plsc.subcore_barrier()
    pltpu.sync_copy(cnt_sh.at[stripe], cnt_hbm.at[c, stripe])


def _make_sc_cnt():
    mesh = plsc.VectorSubcoreMesh(core_axis_name="c", subcore_axis_name="s")
    return pl.kernel(
        _sc_cnt_body,
        out_type=jax.ShapeDtypeStruct((NC, NP, CNTW), jnp.float32),
        mesh=mesh,
        scratch_types=[
            pltpu.VMEM_SHARED((NP, CNTW), jnp.float32),
            pltpu.VMEM((CHC, LANE), jnp.int32),
            pltpu.VMEM((LANE, CNTW), jnp.float32),
        ],
        name="sc_edge_cnt",
    )


BLK = 1024
GRID = NP // BLK


def _tc_pre_body(x_ref, wl_ref, wr_ref, b_ref, z_ref, r_ref):
    x = x_ref[...]
    z_ref[...] = jnp.dot(x, wl_ref[...], preferred_element_type=jnp.float32,
                      precision=lax.Precision.HIGHEST)
    r_ref[...] = (jnp.dot(x, wr_ref[...], preferred_element_type=jnp.float32,
                      precision=lax.Precision.HIGHEST)
                  + b_ref[...])


def _tc_mid_body(aggp_ref, inv_ref, r0_ref, wl_ref, wr_ref, b_ref, z_ref, r_ref):
    agg = aggp_ref[0] + aggp_ref[1]
    x1 = jnp.maximum(agg * inv_ref[...] + r0_ref[...], 0.0)
    z_ref[...] = jnp.dot(x1, wl_ref[...], preferred_element_type=jnp.float32,
                      precision=lax.Precision.HIGHEST)
    r_ref[...] = (jnp.dot(x1, wr_ref[...], preferred_element_type=jnp.float32,
                      precision=lax.Precision.HIGHEST)
                  + b_ref[...])


def _tc_out_body(aggp_ref, inv_ref, r_ref, o_ref):
    agg = aggp_ref[0] + aggp_ref[1]
    o_ref[...] = jnp.maximum(agg * inv_ref[...] + r_ref[...], 0.0)


_row_spec = pl.BlockSpec((BLK, H), lambda i: (i, 0))
_mat_spec = pl.BlockSpec((H, H), lambda i: (0, 0))
_bias_spec = pl.BlockSpec((1, H), lambda i: (0, 0))
_pair_spec = pl.BlockSpec((NC, BLK, H), lambda i: (0, i, 0))

_tc_pre = pl.pallas_call(
    _tc_pre_body,
    grid=(GRID,),
    in_specs=[_row_spec, _mat_spec, _mat_spec, _bias_spec],
    out_specs=[_row_spec, _row_spec],
    out_shape=[jax.ShapeDtypeStruct((NP, H), jnp.float32)] * 2,
)

_tc_mid = pl.pallas_call(
    _tc_mid_body,
    grid=(GRID,),
    in_specs=[_pair_spec, _row_spec, _row_spec, _mat_spec, _mat_spec, _bias_spec],
    out_specs=[_row_spec, _row_spec],
    out_shape=[jax.ShapeDtypeStruct((NP, H), jnp.float32)] * 2,
)

_tc_out = pl.pallas_call(
    _tc_out_body,
    grid=(GRID,),
    in_specs=[_pair_spec, _row_spec, _row_spec],
    out_specs=_row_spec,
    out_shape=jax.ShapeDtypeStruct((NP, H), jnp.float32),
)


def kernel(edge_index, user_table, item_table, Wl0, bl0, Wr0, Wl1, bl1, Wr1):
    x = jnp.concatenate(
        [user_table, item_table,
         jnp.zeros((NP - N, H), jnp.float32)], axis=0)

    pad_idx = (jnp.arange(EP - E, dtype=jnp.int32) % (NP - N)) + N
    pad = jnp.broadcast_to(pad_idx, (2, EP - E))
    ep = jnp.concatenate([edge_index, pad], axis=1).reshape(2, ROWS, LANE)
    ea, eb = ep[0], ep[1]

    bl0_2 = bl0.reshape(1, H)
    bl1_2 = bl1.reshape(1, H)

    sc_agg = _make_sc_agg()
    sc_cnt = _make_sc_cnt()
    zeros_c = jnp.zeros((ROWS_PER_SUB, H), jnp.float32)
    zeros_cnt = jnp.zeros((ROWS_PER_SUB, CNTW), jnp.float32)
    ones_c = jnp.ones((LANE, CNTW), jnp.float32)

    cntp = sc_cnt(ea, eb, zeros_cnt, ones_c)
    cnt = cntp[0, :, 0] + cntp[1, :, 0]
    inv2d = jnp.broadcast_to((1.0 / jnp.maximum(cnt, 1.0))[:, None], (NP, H))

    z0, r0 = _tc_pre(x, Wl0, Wr0, bl0_2)
    aggp0 = sc_agg(ea, eb, z0, zeros_c)

    z1, r1 = _tc_mid(aggp0, inv2d, r0, Wl1, Wr1, bl1_2)
    aggp1 = sc_agg(ea, eb, z1, zeros_c)
    out = _tc_out(aggp1, inv2d, r1)

    return out[:N]


# speedup vs baseline: 1.0759x; 1.0703x over previous
import functools

import jax
import jax.numpy as jnp
from jax import lax
from jax.experimental import pallas as pl
from jax.experimental.pallas import tpu as pltpu
from jax.experimental.pallas import tpu_sc as plsc

N = 10000
NP = 10240
H = 128
E = 320000
LANE = 128
ROWS = 2560
EP = ROWS * LANE
NC, NS = 2, 16
NW = NC * NS
RPW = ROWS // NW
IDXB = 16
CNTW = 128
ROWS_PER_SUB = NP // NS


def _sc_agg_body(ea, eb, z_hbm, zeros_hbm, agg_hbm,
                 agg_sh, idx_g, idx_s, rows_v, sem0, sem1):
    sems = (sem0, sem1)
    c = lax.axis_index("c")
    s = lax.axis_index("s")
    stripe = pl.ds(s * ROWS_PER_SUB, ROWS_PER_SUB)

    pltpu.sync_copy(zeros_hbm, agg_sh.at[stripe])
    plsc.subcore_barrier()

    base_row = c * (ROWS // NC) + s * RPW

    def _scatter(j):
        pltpu.sync_copy(rows_v.at[pl.ds((j % 2) * LANE, LANE)],
                        agg_sh.at[idx_s.at[j]], add=True)

    def _run_pass(gather_e, scatter_e):
        def _block(b, _):
            r0 = base_row + b * IDXB
            pltpu.sync_copy(gather_e.at[pl.ds(r0, IDXB)], idx_g)
            pltpu.sync_copy(scatter_e.at[pl.ds(r0, IDXB)], idx_s)
            cps = [None, None]
            for j in range(IDXB):
                cp = pltpu.make_async_copy(
                    z_hbm.at[idx_g.at[j]],
                    rows_v.at[pl.ds((j % 2) * LANE, LANE)],
                    sems[j % 2],
                )
                cp.start()
                cps[j % 2] = cp
                if j > 0:
                    cps[(j - 1) % 2].wait()
                    _scatter(j - 1)
            cps[(IDXB - 1) % 2].wait()
            _scatter(IDXB - 1)
            return 0

        lax.fori_loop(0, RPW // IDXB, _block, 0)

    _run_pass(ea, eb)
    _run_pass(eb, ea)

    plsc.subcore_barrier()

    pltpu.sync_copy(agg_sh.at[stripe], agg_hbm.at[c, stripe])


def _make_sc_agg():
    mesh = plsc.VectorSubcoreMesh(core_axis_name="c", subcore_axis_name="s")
    return pl.kernel(
        _sc_agg_body,
        out_type=jax.ShapeDtypeStruct((NC, NP, H), jnp.float32),
        mesh=mesh,
        scratch_types=[
            pltpu.VMEM_SHARED((NP, H), jnp.float32),
            pltpu.VMEM((IDXB, LANE), jnp.int32),
            pltpu.VMEM((IDXB, LANE), jnp.int32),
            pltpu.VMEM((2 * LANE, H), jnp.float32),
            pltpu.SemaphoreType.DMA,
            pltpu.SemaphoreType.DMA,
        ],
        name="sc_edge_agg",
    )


CHC = 8


def _sc_cnt_body(ea, eb, zeros_cnt_hbm, ones_hbm, cnt_hbm,
                 cnt_sh, idx_s, ones_v):
    c = lax.axis_index("c")
    s = lax.axis_index("s")
    stripe = pl.ds(s * ROWS_PER_SUB, ROWS_PER_SUB)

    pltpu.sync_copy(zeros_cnt_hbm, cnt_sh.at[stripe])
    pltpu.sync_copy(ones_hbm, ones_v)
    plsc.subcore_barrier()

    base_row = c * (ROWS // NC) + s * RPW

    def _run_pass(scatter_e):
        def _chunk(i, _):
            r0 = base_row + i * CHC
            pltpu.sync_copy(scatter_e.at[pl.ds(r0, CHC)], idx_s)
            for j in range(CHC):
                pltpu.sync_copy(ones_v, cnt_sh.at[idx_s.at[j]], add=True)
            return 0

        lax.fori_loop(0, RPW // CHC, _chunk, 0)

    _run_pass(eb)
    _run_pass(ea)

    plsc.subcore_barrier()
    pltpu.sync_copy(cnt_sh.at[stripe], cnt_hbm.at[c, stripe])


def _make_sc_cnt():
    mesh = plsc.VectorSubcoreMesh(core_axis_name="c", subcore_axis_name="s")
    return pl.kernel(
        _sc_cnt_body,
        out_type=jax.ShapeDtypeStruct((NC, NP, CNTW), jnp.float32),
        mesh=mesh,
        scratch_types=[
            pltpu.VMEM_SHARED((NP, CNTW), jnp.float32),
            pltpu.VMEM((CHC, LANE), jnp.int32),
            pltpu.VMEM((LANE, CNTW), jnp.float32),
        ],
        name="sc_edge_cnt",
    )


BLK = 1024
GRID = NP // BLK


def _tc_pre_body(x_ref, wl_ref, wr_ref, b_ref, z_ref, r_ref):
    x = x_ref[...]
    z_ref[...] = jnp.dot(x, wl_ref[...], preferred_element_type=jnp.float32,
                      precision=lax.Precision.HIGHEST)
    r_ref[...] = (jnp.dot(x, wr_ref[...], preferred_element_type=jnp.float32,
                      precision=lax.Precision.HIGHEST)
                  + b_ref[...])


def _tc_mid_body(aggp_ref, inv_ref, r0_ref, wl_ref, wr_ref, b_ref, z_ref, r_ref):
    agg = aggp_ref[0] + aggp_ref[1]
    x1 = jnp.maximum(agg * inv_ref[...] + r0_ref[...], 0.0)
    z_ref[...] = jnp.dot(x1, wl_ref[...], preferred_element_type=jnp.float32,
                      precision=lax.Precision.HIGHEST)
    r_ref[...] = (jnp.dot(x1, wr_ref[...], preferred_element_type=jnp.float32,
                      precision=lax.Precision.HIGHEST)
                  + b_ref[...])


def _tc_out_body(aggp_ref, inv_ref, r_ref, o_ref):
    agg = aggp_ref[0] + aggp_ref[1]
    o_ref[...] = jnp.maximum(agg * inv_ref[...] + r_ref[...], 0.0)


_row_spec = pl.BlockSpec((BLK, H), lambda i: (i, 0))
_mat_spec = pl.BlockSpec((H, H), lambda i: (0, 0))
_bias_spec = pl.BlockSpec((1, H), lambda i: (0, 0))
_pair_spec = pl.BlockSpec((NC, BLK, H), lambda i: (0, i, 0))

_tc_pre = pl.pallas_call(
    _tc_pre_body,
    grid=(GRID,),
    in_specs=[_row_spec, _mat_spec, _mat_spec, _bias_spec],
    out_specs=[_row_spec, _row_spec],
    out_shape=[jax.ShapeDtypeStruct((NP, H), jnp.float32)] * 2,
)

_tc_mid = pl.pallas_call(
    _tc_mid_body,
    grid=(GRID,),
    in_specs=[_pair_spec, _row_spec, _row_spec, _mat_spec, _mat_spec, _bias_spec],
    out_specs=[_row_spec, _row_spec],
    out_shape=[jax.ShapeDtypeStruct((NP, H), jnp.float32)] * 2,
)

_tc_out = pl.pallas_call(
    _tc_out_body,
    grid=(GRID,),
    in_specs=[_pair_spec, _row_spec, _row_spec],
    out_specs=_row_spec,
    out_shape=jax.ShapeDtypeStruct((NP, H), jnp.float32),
)


def kernel(edge_index, user_table, item_table, Wl0, bl0, Wr0, Wl1, bl1, Wr1):
    x = jnp.concatenate(
        [user_table, item_table,
         jnp.zeros((NP - N, H), jnp.float32)], axis=0)

    pad_idx = (jnp.arange(EP - E, dtype=jnp.int32) % (NP - N)) + N
    pad = jnp.broadcast_to(pad_idx, (2, EP - E))
    ep = jnp.concatenate([edge_index, pad], axis=1).reshape(2, ROWS, LANE)
    ea, eb = ep[0], ep[1]

    bl0_2 = bl0.reshape(1, H)
    bl1_2 = bl1.reshape(1, H)

    sc_agg = _make_sc_agg()
    sc_cnt = _make_sc_cnt()
    zeros_c = jnp.zeros((ROWS_PER_SUB, H), jnp.float32)
    zeros_cnt = jnp.zeros((ROWS_PER_SUB, CNTW), jnp.float32)
    ones_c = jnp.ones((LANE, CNTW), jnp.float32)

    cntp = sc_cnt(ea, eb, zeros_cnt, ones_c)
    cnt = cntp[0, :, 0] + cntp[1, :, 0]
    inv2d = jnp.broadcast_to((1.0 / jnp.maximum(cnt, 1.0))[:, None], (NP, H))

    z0, r0 = _tc_pre(x, Wl0, Wr0, bl0_2)
    aggp0 = sc_agg(ea, eb, z0, zeros_c)

    z1, r1 = _tc_mid(aggp0, inv2d, r0, Wl1, Wr1, bl1_2)
    aggp1 = sc_agg(ea, eb, z1, zeros_c)
    out = _tc_out(aggp1, inv2d, r1)

    return out[:N]


# speedup vs baseline: 1.1290x; 1.0494x over previous
import functools

import jax
import jax.numpy as jnp
from jax import lax
from jax.experimental import pallas as pl
from jax.experimental.pallas import tpu as pltpu
from jax.experimental.pallas import tpu_sc as plsc

N = 10000
NP = 10240
H = 128
E = 320000
LANE = 128
ROWS = 2560
EP = ROWS * LANE
NC, NS = 2, 16
NW = NC * NS
RPW = ROWS // NW
IDXB = 40
CNTW = 128
ROWS_PER_SUB = NP // NS


def _sc_agg_body(ea, eb, z_hbm, zeros_hbm, agg_hbm,
                 agg_sh, idx_g, idx_s, rows_v, sem0, sem1):
    sems = (sem0, sem1)
    c = lax.axis_index("c")
    s = lax.axis_index("s")
    stripe = pl.ds(s * ROWS_PER_SUB, ROWS_PER_SUB)

    pltpu.sync_copy(zeros_hbm, agg_sh.at[stripe])
    plsc.subcore_barrier()

    base_row = c * (ROWS // NC) + s * RPW

    def _scatter(j):
        pltpu.sync_copy(rows_v.at[pl.ds((j % 2) * LANE, LANE)],
                        agg_sh.at[idx_s.at[j]], add=True)

    def _run_pass(gather_e, scatter_e):
        def _block(b, _):
            r0 = base_row + b * IDXB
            pltpu.sync_copy(gather_e.at[pl.ds(r0, IDXB)], idx_g)
            pltpu.sync_copy(scatter_e.at[pl.ds(r0, IDXB)], idx_s)
            cps = [None, None]
            for j in range(IDXB):
                cp = pltpu.make_async_copy(
                    z_hbm.at[idx_g.at[j]],
                    rows_v.at[pl.ds((j % 2) * LANE, LANE)],
                    sems[j % 2],
                )
                cp.start()
                cps[j % 2] = cp
                if j > 0:
                    cps[(j - 1) % 2].wait()
                    _scatter(j - 1)
            cps[(IDXB - 1) % 2].wait()
            _scatter(IDXB - 1)
            return 0

        lax.fori_loop(0, RPW // IDXB, _block, 0)

    _run_pass(ea, eb)
    _run_pass(eb, ea)

    plsc.subcore_barrier()

    pltpu.sync_copy(agg_sh.at[stripe], agg_hbm.at[c, stripe])


def _make_sc_agg():
    mesh = plsc.VectorSubcoreMesh(core_axis_name="c", subcore_axis_name="s")
    return pl.kernel(
        _sc_agg_body,
        out_type=jax.ShapeDtypeStruct((NC, NP, H), jnp.float32),
        mesh=mesh,
        scratch_types=[
            pltpu.VMEM_SHARED((NP, H), jnp.float32),
            pltpu.VMEM((IDXB, LANE), jnp.int32),
            pltpu.VMEM((IDXB, LANE), jnp.int32),
            pltpu.VMEM((2 * LANE, H), jnp.float32),
            pltpu.SemaphoreType.DMA,
            pltpu.SemaphoreType.DMA,
        ],
        name="sc_edge_agg",
    )


CHC = 8


def _sc_cnt_body(ea, eb, zeros_cnt_hbm, ones_hbm, cnt_hbm,
                 cnt_sh, idx_s, ones_v):
    c = lax.axis_index("c")
    s = lax.axis_index("s")
    stripe = pl.ds(s * ROWS_PER_SUB, ROWS_PER_SUB)

    pltpu.sync_copy(zeros_cnt_hbm, cnt_sh.at[stripe])
    pltpu.sync_copy(ones_hbm, ones_v)
    plsc.subcore_barrier()

    base_row = c * (ROWS // NC) + s * RPW

    def _run_pass(scatter_e):
        def _chunk(i, _):
            r0 = base_row + i * CHC
            pltpu.sync_copy(scatter_e.at[pl.ds(r0, CHC)], idx_s)
            for j in range(CHC):
                pltpu.sync_copy(ones_v, cnt_sh.at[idx_s.at[j]], add=True)
            return 0

        lax.fori_loop(0, RPW // CHC, _chunk, 0)

    _run_pass(eb)
    _run_pass(ea)

    plsc.subcore_barrier()
    pltpu.sync_copy(cnt_sh.at[stripe], cnt_hbm.at[c, stripe])


def _make_sc_cnt():
    mesh = plsc.VectorSubcoreMesh(core_axis_name="c", subcore_axis_name="s")
    return pl.kernel(
        _sc_cnt_body,
        out_type=jax.ShapeDtypeStruct((NC, NP, CNTW), jnp.float32),
        mesh=mesh,
        scratch_types=[
            pltpu.VMEM_SHARED((NP, CNTW), jnp.float32),
            pltpu.VMEM((CHC, LANE), jnp.int32),
            pltpu.VMEM((LANE, CNTW), jnp.float32),
        ],
        name="sc_edge_cnt",
    )


BLK = 1024
GRID = NP // BLK


def _tc_pre_body(x_ref, wl_ref, wr_ref, b_ref, z_ref, r_ref):
    x = x_ref[...]
    z_ref[...] = jnp.dot(x, wl_ref[...], preferred_element_type=jnp.float32,
                      precision=lax.Precision.HIGHEST)
    r_ref[...] = (jnp.dot(x, wr_ref[...], preferred_element_type=jnp.float32,
                      precision=lax.Precision.HIGHEST)
                  + b_ref[...])


def _tc_mid_body(aggp_ref, inv_ref, r0_ref, wl_ref, wr_ref, b_ref, z_ref, r_ref):
    agg = aggp_ref[0] + aggp_ref[1]
    x1 = jnp.maximum(agg * inv_ref[...] + r0_ref[...], 0.0)
    z_ref[...] = jnp.dot(x1, wl_ref[...], preferred_element_type=jnp.float32,
                      precision=lax.Precision.HIGHEST)
    r_ref[...] = (jnp.dot(x1, wr_ref[...], preferred_element_type=jnp.float32,
                      precision=lax.Precision.HIGHEST)
                  + b_ref[...])


def _tc_out_body(aggp_ref, inv_ref, r_ref, o_ref):
    agg = aggp_ref[0] + aggp_ref[1]
    o_ref[...] = jnp.maximum(agg * inv_ref[...] + r_ref[...], 0.0)


_row_spec = pl.BlockSpec((BLK, H), lambda i: (i, 0))
_mat_spec = pl.BlockSpec((H, H), lambda i: (0, 0))
_bias_spec = pl.BlockSpec((1, H), lambda i: (0, 0))
_pair_spec = pl.BlockSpec((NC, BLK, H), lambda i: (0, i, 0))

_tc_pre = pl.pallas_call(
    _tc_pre_body,
    grid=(GRID,),
    in_specs=[_row_spec, _mat_spec, _mat_spec, _bias_spec],
    out_specs=[_row_spec, _row_spec],
    out_shape=[jax.ShapeDtypeStruct((NP, H), jnp.float32)] * 2,
)

_tc_mid = pl.pallas_call(
    _tc_mid_body,
    grid=(GRID,),
    in_specs=[_pair_spec, _row_spec, _row_spec, _mat_spec, _mat_spec, _bias_spec],
    out_specs=[_row_spec, _row_spec],
    out_shape=[jax.ShapeDtypeStruct((NP, H), jnp.float32)] * 2,
)

_tc_out = pl.pallas_call(
    _tc_out_body,
    grid=(GRID,),
    in_specs=[_pair_spec, _row_spec, _row_spec],
    out_specs=_row_spec,
    out_shape=jax.ShapeDtypeStruct((NP, H), jnp.float32),
)


def kernel(edge_index, user_table, item_table, Wl0, bl0, Wr0, Wl1, bl1, Wr1):
    x = jnp.concatenate(
        [user_table, item_table,
         jnp.zeros((NP - N, H), jnp.float32)], axis=0)

    pad_idx = (jnp.arange(EP - E, dtype=jnp.int32) % (NP - N)) + N
    pad = jnp.broadcast_to(pad_idx, (2, EP - E))
    ep = jnp.concatenate([edge_index, pad], axis=1).reshape(2, ROWS, LANE)
    ea, eb = ep[0], ep[1]

    bl0_2 = bl0.reshape(1, H)
    bl1_2 = bl1.reshape(1, H)

    sc_agg = _make_sc_agg()
    sc_cnt = _make_sc_cnt()
    zeros_c = jnp.zeros((ROWS_PER_SUB, H), jnp.float32)
    zeros_cnt = jnp.zeros((ROWS_PER_SUB, CNTW), jnp.float32)
    ones_c = jnp.ones((LANE, CNTW), jnp.float32)

    cntp = sc_cnt(ea, eb, zeros_cnt, ones_c)
    cnt = cntp[0, :, 0] + cntp[1, :, 0]
    inv2d = jnp.broadcast_to((1.0 / jnp.maximum(cnt, 1.0))[:, None], (NP, H))

    z0, r0 = _tc_pre(x, Wl0, Wr0, bl0_2)
    aggp0 = sc_agg(ea, eb, z0, zeros_c)

    z1, r1 = _tc_mid(aggp0, inv2d, r0, Wl1, Wr1, bl1_2)
    aggp1 = sc_agg(ea, eb, z1, zeros_c)
    out = _tc_out(aggp1, inv2d, r1)

    return out[:N]


# speedup vs baseline: 1.1444x; 1.0136x over previous
import functools

import jax
import jax.numpy as jnp
from jax import lax
from jax.experimental import pallas as pl
from jax.experimental.pallas import tpu as pltpu
from jax.experimental.pallas import tpu_sc as plsc

N = 10000
NP = 10240
H = 128
E = 320000
LANE = 128
ROWS = 2560
EP = ROWS * LANE
NC, NS = 2, 16
NW = NC * NS
RPW = ROWS // NW
IDXB = 40
CNTW = 128
ROWS_PER_SUB = NP // NS


def _sc_agg_body(ea, eb, z_hbm, zeros_hbm, agg_hbm,
                 agg_sh, idx_g, idx_s, rows_v, sem0, sem1):
    sems = (sem0, sem1)
    c = lax.axis_index("c")
    s = lax.axis_index("s")
    stripe = pl.ds(s * ROWS_PER_SUB, ROWS_PER_SUB)

    pltpu.sync_copy(zeros_hbm, agg_sh.at[stripe])
    plsc.subcore_barrier()

    base_row = c * (ROWS // NC) + s * RPW

    def _scatter(j):
        pltpu.sync_copy(rows_v.at[pl.ds((j % 2) * LANE, LANE)],
                        agg_sh.at[idx_s.at[j]], add=True)

    def _run_pass(gather_e, scatter_e):
        def _block(b, _):
            r0 = base_row + b * IDXB
            pltpu.sync_copy(gather_e.at[pl.ds(r0, IDXB)], idx_g)
            pltpu.sync_copy(scatter_e.at[pl.ds(r0, IDXB)], idx_s)
            cps = [None, None]
            for j in range(IDXB):
                cp = pltpu.make_async_copy(
                    z_hbm.at[idx_g.at[j]],
                    rows_v.at[pl.ds((j % 2) * LANE, LANE)],
                    sems[j % 2],
                )
                cp.start()
                cps[j % 2] = cp
                if j > 0:
                    cps[(j - 1) % 2].wait()
                    _scatter(j - 1)
            cps[(IDXB - 1) % 2].wait()
            _scatter(IDXB - 1)
            return 0

        lax.fori_loop(0, RPW // IDXB, _block, 0)

    _run_pass(ea, eb)
    _run_pass(eb, ea)

    plsc.subcore_barrier()

    pltpu.sync_copy(agg_sh.at[stripe], agg_hbm.at[c, stripe])


def _make_sc_agg():
    mesh = plsc.VectorSubcoreMesh(core_axis_name="c", subcore_axis_name="s")
    return pl.kernel(
        _sc_agg_body,
        out_type=jax.ShapeDtypeStruct((NC, NP, H), jnp.float32),
        mesh=mesh,
        scratch_types=[
            pltpu.VMEM_SHARED((NP, H), jnp.float32),
            pltpu.VMEM((IDXB, LANE), jnp.int32),
            pltpu.VMEM((IDXB, LANE), jnp.int32),
            pltpu.VMEM((2 * LANE, H), jnp.float32),
            pltpu.SemaphoreType.DMA,
            pltpu.SemaphoreType.DMA,
        ],
        name="sc_edge_agg",
    )


CHC = 40


def _sc_cnt_body(ea, eb, zeros_cnt_hbm, ones_hbm, cnt_hbm,
                 cnt_sh, idx_s, ones_v):
    c = lax.axis_index("c")
    s = lax.axis_index("s")
    stripe = pl.ds(s * ROWS_PER_SUB, ROWS_PER_SUB)

    pltpu.sync_copy(zeros_cnt_hbm, cnt_sh.at[stripe])
    pltpu.sync_copy(ones_hbm, ones_v)
    plsc.subcore_barrier()

    base_row = c * (ROWS // NC) + s * RPW

    def _run_pass(scatter_e):
        def _chunk(i, _):
            r0 = base_row + i * CHC
            pltpu.sync_copy(scatter_e.at[pl.ds(r0, CHC)], idx_s)
            for j in range(CHC):
                pltpu.sync_copy(ones_v, cnt_sh.at[idx_s.at[j]], add=True)
            return 0

        lax.fori_loop(0, RPW // CHC, _chunk, 0)

    _run_pass(eb)
    _run_pass(ea)

    plsc.subcore_barrier()
    pltpu.sync_copy(cnt_sh.at[stripe], cnt_hbm.at[c, stripe])


def _make_sc_cnt():
    mesh = plsc.VectorSubcoreMesh(core_axis_name="c", subcore_axis_name="s")
    return pl.kernel(
        _sc_cnt_body,
        out_type=jax.ShapeDtypeStruct((NC, NP, CNTW), jnp.float32),
        mesh=mesh,
        scratch_types=[
            pltpu.VMEM_SHARED((NP, CNTW), jnp.float32),
            pltpu.VMEM((CHC, LANE), jnp.int32),
            pltpu.VMEM((LANE, CNTW), jnp.float32),
        ],
        name="sc_edge_cnt",
    )


BLK = 1024
GRID = NP // BLK


def _tc_pre_body(x_ref, wl_ref, wr_ref, b_ref, z_ref, r_ref):
    x = x_ref[...]
    z_ref[...] = jnp.dot(x, wl_ref[...], preferred_element_type=jnp.float32,
                      precision=lax.Precision.HIGHEST)
    r_ref[...] = (jnp.dot(x, wr_ref[...], preferred_element_type=jnp.float32,
                      precision=lax.Precision.HIGHEST)
                  + b_ref[...])


def _tc_mid_body(aggp_ref, inv_ref, r0_ref, wl_ref, wr_ref, b_ref, z_ref, r_ref):
    agg = aggp_ref[0] + aggp_ref[1]
    x1 = jnp.maximum(agg * inv_ref[...] + r0_ref[...], 0.0)
    z_ref[...] = jnp.dot(x1, wl_ref[...], preferred_element_type=jnp.float32,
                      precision=lax.Precision.HIGHEST)
    r_ref[...] = (jnp.dot(x1, wr_ref[...], preferred_element_type=jnp.float32,
                      precision=lax.Precision.HIGHEST)
                  + b_ref[...])


def _tc_out_body(aggp_ref, inv_ref, r_ref, o_ref):
    agg = aggp_ref[0] + aggp_ref[1]
    o_ref[...] = jnp.maximum(agg * inv_ref[...] + r_ref[...], 0.0)


_row_spec = pl.BlockSpec((BLK, H), lambda i: (i, 0))
_mat_spec = pl.BlockSpec((H, H), lambda i: (0, 0))
_bias_spec = pl.BlockSpec((1, H), lambda i: (0, 0))
_pair_spec = pl.BlockSpec((NC, BLK, H), lambda i: (0, i, 0))

_tc_pre = pl.pallas_call(
    _tc_pre_body,
    grid=(GRID,),
    in_specs=[_row_spec, _mat_spec, _mat_spec, _bias_spec],
    out_specs=[_row_spec, _row_spec],
    out_shape=[jax.ShapeDtypeStruct((NP, H), jnp.float32)] * 2,
)

_tc_mid = pl.pallas_call(
    _tc_mid_body,
    grid=(GRID,),
    in_specs=[_pair_spec, _row_spec, _row_spec, _mat_spec, _mat_spec, _bias_spec],
    out_specs=[_row_spec, _row_spec],
    out_shape=[jax.ShapeDtypeStruct((NP, H), jnp.float32)] * 2,
)

_tc_out = pl.pallas_call(
    _tc_out_body,
    grid=(GRID,),
    in_specs=[_pair_spec, _row_spec, _row_spec],
    out_specs=_row_spec,
    out_shape=jax.ShapeDtypeStruct((NP, H), jnp.float32),
)


def kernel(edge_index, user_table, item_table, Wl0, bl0, Wr0, Wl1, bl1, Wr1):
    x = jnp.concatenate(
        [user_table, item_table,
         jnp.zeros((NP - N, H), jnp.float32)], axis=0)

    pad_idx = (jnp.arange(EP - E, dtype=jnp.int32) % (NP - N)) + N
    pad = jnp.broadcast_to(pad_idx, (2, EP - E))
    ep = jnp.concatenate([edge_index, pad], axis=1).reshape(2, ROWS, LANE)
    ea, eb = ep[0], ep[1]

    bl0_2 = bl0.reshape(1, H)
    bl1_2 = bl1.reshape(1, H)

    sc_agg = _make_sc_agg()
    sc_cnt = _make_sc_cnt()
    zeros_c = jnp.zeros((ROWS_PER_SUB, H), jnp.float32)
    zeros_cnt = jnp.zeros((ROWS_PER_SUB, CNTW), jnp.float32)
    ones_c = jnp.ones((LANE, CNTW), jnp.float32)

    cntp = sc_cnt(ea, eb, zeros_cnt, ones_c)
    cnt = cntp[0, :, 0] + cntp[1, :, 0]
    inv2d = jnp.broadcast_to((1.0 / jnp.maximum(cnt, 1.0))[:, None], (NP, H))

    z0, r0 = _tc_pre(x, Wl0, Wr0, bl0_2)
    aggp0 = sc_agg(ea, eb, z0, zeros_c)

    z1, r1 = _tc_mid(aggp0, inv2d, r0, Wl1, Wr1, bl1_2)
    aggp1 = sc_agg(ea, eb, z1, zeros_c)
    out = _tc_out(aggp1, inv2d, r1)

    return out[:N]
